# Initial kernel scaffold; baseline (speedup 1.0000x reference)
import jax
import jax.numpy as jnp
from jax.experimental import pallas as pl


def kernel(x, edge_index, W1, b1, W2, b2, original_size):
    raise NotImplementedError("write your pallas kernel here")


# speedup vs baseline: 13.3508x; 13.3508x over previous
import functools

import jax
import jax.numpy as jnp
from jax import lax
from jax.experimental import pallas as pl
from jax.experimental.pallas import tpu as pltpu
from jax.experimental.pallas import tpu_sc as plsc

N = 10000
D = 128
E = 320000
NC = 2
NS = 16
EDGES_PER_TILE = E // (NC * NS)
WIN = 80
NWIN = EDGES_PER_TILE // WIN
CH = 624
LAST_START = CH * (NS - 1)
LAST = N - LAST_START
BN = 1000
GRID = N // BN

_mesh = plsc.VectorSubcoreMesh(
    core_axis_name="c", subcore_axis_name="s", num_cores=NC, num_subcores=NS
)


def _chunked_row_copy(s, copy_fn):
    @pl.when(s < NS - 1)
    def _():
        copy_fn(pl.ds(pl.multiple_of(s * CH, 8), CH), CH)

    @pl.when(s == NS - 1)
    def _():
        copy_fn(pl.ds(LAST_START, LAST), LAST)


@functools.partial(
    pl.kernel,
    out_type=[
        jax.ShapeDtypeStruct((N,), jnp.float32),
        jax.ShapeDtypeStruct((N,), jnp.float32),
    ],
    mesh=_mesh,
    scratch_types=[
        pltpu.VMEM((WIN,), jnp.int32),
        pltpu.VMEM((WIN,), jnp.float32),
        pltpu.VMEM((LAST,), jnp.float32),
        pltpu.VMEM_SHARED((N,), jnp.float32),
    ],
)
def _deg_sc(dst_hbm, ones_hbm, z1d_hbm, deg0_out, deg1_out, idx_v, ones_v, vbuf, acc):
    c = lax.axis_index("c")
    s = lax.axis_index("s")

    def init_chunk(r, n):
        pltpu.sync_copy(z1d_hbm.at[r], vbuf.at[pl.ds(0, n)])
        pltpu.sync_copy(vbuf.at[pl.ds(0, n)], acc.at[r])

    _chunked_row_copy(s, init_chunk)
    pltpu.sync_copy(ones_hbm, ones_v)
    plsc.subcore_barrier()
    tile_base = (c * NS + s) * EDGES_PER_TILE

    def body(w, carry):
        base = pl.multiple_of(tile_base + w * WIN, 8)
        pltpu.sync_copy(dst_hbm.at[pl.ds(base, WIN)], idx_v)
        pltpu.sync_copy(ones_v, acc.at[idx_v], add=True)
        return carry

    lax.fori_loop(0, NWIN, body, 0)
    plsc.subcore_barrier()

    def out_chunk(out_ref, r, n):
        pltpu.sync_copy(acc.at[r], vbuf.at[pl.ds(0, n)])
        pltpu.sync_copy(vbuf.at[pl.ds(0, n)], out_ref.at[r])

    @pl.when(c == 0)
    def _():
        _chunked_row_copy(s, lambda r, n: out_chunk(deg0_out, r, n))

    @pl.when(c != 0)
    def _():
        _chunked_row_copy(s, lambda r, n: out_chunk(deg1_out, r, n))


@functools.partial(
    pl.kernel,
    out_type=jax.ShapeDtypeStruct((NC, N, D), jnp.float32),
    mesh=_mesh,
    scratch_types=[
        pltpu.VMEM((WIN,), jnp.int32),
        pltpu.VMEM((WIN,), jnp.int32),
        pltpu.VMEM((WIN, D), jnp.float32),
        pltpu.VMEM_SHARED((N, D), jnp.float32),
        pltpu.SemaphoreType.DMA,
    ],
)
def _spmm_sc(src_hbm, dst_hbm, g_hbm, zrows_hbm, parts_out, src_v, dst_v, rows_v, acc, sem):
    c = lax.axis_index("c")
    s = lax.axis_index("s")

    @pl.when(c == 0)
    def _():
        _chunked_row_copy(s, lambda r, n: pltpu.sync_copy(g_hbm.at[r], acc.at[r]))

    @pl.when(c != 0)
    def _():
        _chunked_row_copy(s, lambda r, n: pltpu.sync_copy(zrows_hbm.at[r], acc.at[r]))

    plsc.subcore_barrier()
    tile_base = (c * NS + s) * EDGES_PER_TILE

    def body(w, carry):
        base = pl.multiple_of(tile_base + w * WIN, 8)
        pltpu.sync_copy(src_hbm.at[pl.ds(base, WIN)], src_v)
        pltpu.sync_copy(dst_hbm.at[pl.ds(base, WIN)], dst_v)
        pltpu.async_copy(g_hbm.at[src_v], rows_v, sem).wait()
        pltpu.sync_copy(rows_v, acc.at[dst_v], add=True)
        return carry

    lax.fori_loop(0, NWIN, body, 0)
    plsc.subcore_barrier()
    _chunked_row_copy(s, lambda r, n: pltpu.sync_copy(acc.at[r], parts_out.at[c, r]))


def _stage1_body(x_ref, w_ref, deg0_ref, deg1_ref, g_ref, dinv_ref):
    d = deg0_ref[...] + deg1_ref[...] + 1.0
    dinv = lax.rsqrt(d)
    h = jnp.dot(x_ref[...], w_ref[...], preferred_element_type=jnp.float32)
    g_ref[...] = h * dinv
    dinv_ref[...] = dinv


def _stage1(x, W1, deg0, deg1):
    return pl.pallas_call(
        _stage1_body,
        grid=(GRID,),
        in_specs=[
            pl.BlockSpec((BN, D), lambda i: (i, 0)),
            pl.BlockSpec((D, D), lambda i: (0, 0)),
            pl.BlockSpec((BN, 1), lambda i: (i, 0)),
            pl.BlockSpec((BN, 1), lambda i: (i, 0)),
        ],
        out_specs=[
            pl.BlockSpec((BN, D), lambda i: (i, 0)),
            pl.BlockSpec((BN, 1), lambda i: (i, 0)),
        ],
        out_shape=[
            jax.ShapeDtypeStruct((N, D), jnp.float32),
            jax.ShapeDtypeStruct((N, 1), jnp.float32),
        ],
    )(x, W1, deg0, deg1)


def _stage2_body(parts_ref, dinv_ref, w_ref, b_ref, g_ref):
    o = dinv_ref[...] * (parts_ref[0] + parts_ref[1]) + b_ref[...]
    h = jnp.dot(o, w_ref[...], preferred_element_type=jnp.float32)
    g_ref[...] = h * dinv_ref[...]


def _stage2(parts, dinv, W2, b1r):
    return pl.pallas_call(
        _stage2_body,
        grid=(GRID,),
        in_specs=[
            pl.BlockSpec((NC, BN, D), lambda i: (0, i, 0)),
            pl.BlockSpec((BN, 1), lambda i: (i, 0)),
            pl.BlockSpec((D, D), lambda i: (0, 0)),
            pl.BlockSpec((1, D), lambda i: (0, 0)),
        ],
        out_specs=pl.BlockSpec((BN, D), lambda i: (i, 0)),
        out_shape=jax.ShapeDtypeStruct((N, D), jnp.float32),
    )(parts, dinv, W2, b1r)


def _stage3_body(parts_ref, dinv_ref, b_ref, out_ref):
    o = dinv_ref[...] * (parts_ref[0] + parts_ref[1]) + b_ref[...]
    m = jnp.max(o, axis=1, keepdims=True)
    ex = jnp.exp(o - m)
    lse = jnp.log(jnp.sum(ex, axis=1, keepdims=True))
    out_ref[...] = o - m - lse


def _stage3(parts, dinv, b2r):
    return pl.pallas_call(
        _stage3_body,
        grid=(GRID,),
        in_specs=[
            pl.BlockSpec((NC, BN, D), lambda i: (0, i, 0)),
            pl.BlockSpec((BN, 1), lambda i: (i, 0)),
            pl.BlockSpec((1, D), lambda i: (0, 0)),
        ],
        out_specs=pl.BlockSpec((BN, D), lambda i: (i, 0)),
        out_shape=jax.ShapeDtypeStruct((N, D), jnp.float32),
    )(parts, dinv, b2r)


def kernel(x, edge_index, W1, b1, W2, b2, original_size):
    ones_win = jnp.ones((WIN,), jnp.float32)
    z1d = jnp.zeros((N,), jnp.float32)
    zrows = jnp.zeros((N, D), jnp.float32)
    src = edge_index[0]
    dst = edge_index[1]

    deg0, deg1 = _deg_sc(dst, ones_win, z1d)
    g1, dinv = _stage1(x, W1, jnp.reshape(deg0, (N, 1)), jnp.reshape(deg1, (N, 1)))
    parts1 = _spmm_sc(src, dst, g1, zrows)
    g2 = _stage2(parts1, dinv, W2, jnp.reshape(b1, (1, D)))
    parts2 = _spmm_sc(src, dst, g2, zrows)
    out = _stage3(parts2, dinv, jnp.reshape(b2, (1, D)))
    return out


# speedup vs baseline: 22.1293x; 1.6575x over previous
import functools

import jax
import jax.numpy as jnp
from jax import lax
from jax.experimental import pallas as pl
from jax.experimental.pallas import tpu as pltpu
from jax.experimental.pallas import tpu_sc as plsc

N = 10000
D = 128
E = 320000
NC = 2
NS = 16
EDGES_PER_TILE = E // (NC * NS)
WIN = 80
NWIN = EDGES_PER_TILE // WIN
CH = 624
LAST_START = CH * (NS - 1)
LAST = N - LAST_START
BN = 1000
GRID = N // BN

_mesh = plsc.VectorSubcoreMesh(
    core_axis_name="c", subcore_axis_name="s", num_cores=NC, num_subcores=NS
)


def _chunked_row_copy(s, copy_fn):
    @pl.when(s < NS - 1)
    def _():
        copy_fn(pl.ds(pl.multiple_of(s * CH, 8), CH), CH)

    @pl.when(s == NS - 1)
    def _():
        copy_fn(pl.ds(LAST_START, LAST), LAST)


@functools.partial(
    pl.kernel,
    out_type=[
        jax.ShapeDtypeStruct((N,), jnp.float32),
        jax.ShapeDtypeStruct((N,), jnp.float32),
    ],
    mesh=_mesh,
    scratch_types=[
        pltpu.VMEM((WIN,), jnp.int32),
        pltpu.VMEM((WIN,), jnp.float32),
        pltpu.VMEM((LAST,), jnp.float32),
        pltpu.VMEM_SHARED((N,), jnp.float32),
    ],
)
def _deg_sc(dst_hbm, ones_hbm, z1d_hbm, deg0_out, deg1_out, idx_v, ones_v, vbuf, acc):
    c = lax.axis_index("c")
    s = lax.axis_index("s")

    def init_chunk(r, n):
        pltpu.sync_copy(z1d_hbm.at[r], vbuf.at[pl.ds(0, n)])
        pltpu.sync_copy(vbuf.at[pl.ds(0, n)], acc.at[r])

    _chunked_row_copy(s, init_chunk)
    pltpu.sync_copy(ones_hbm, ones_v)
    plsc.subcore_barrier()
    tile_base = (c * NS + s) * EDGES_PER_TILE

    def body(w, carry):
        base = pl.multiple_of(tile_base + w * WIN, 8)
        pltpu.sync_copy(dst_hbm.at[pl.ds(base, WIN)], idx_v)
        pltpu.sync_copy(ones_v, acc.at[idx_v], add=True)
        return carry

    lax.fori_loop(0, NWIN, body, 0)
    plsc.subcore_barrier()

    def out_chunk(out_ref, r, n):
        pltpu.sync_copy(acc.at[r], vbuf.at[pl.ds(0, n)])
        pltpu.sync_copy(vbuf.at[pl.ds(0, n)], out_ref.at[r])

    @pl.when(c == 0)
    def _():
        _chunked_row_copy(s, lambda r, n: out_chunk(deg0_out, r, n))

    @pl.when(c != 0)
    def _():
        _chunked_row_copy(s, lambda r, n: out_chunk(deg1_out, r, n))


NBUF = 3


@functools.partial(
    pl.kernel,
    out_type=jax.ShapeDtypeStruct((NC, N, D), jnp.float32),
    mesh=_mesh,
    scratch_types=[
        [pltpu.VMEM((WIN,), jnp.int32)] * NBUF,
        [pltpu.VMEM((WIN,), jnp.int32)] * NBUF,
        [pltpu.VMEM((WIN, D), jnp.float32)] * NBUF,
        pltpu.VMEM_SHARED((N, D), jnp.float32),
        [pltpu.SemaphoreType.DMA] * NBUF,
        [pltpu.SemaphoreType.DMA] * NBUF,
        [pltpu.SemaphoreType.DMA] * NBUF,
    ],
)
def _spmm_sc(src_hbm, dst_hbm, g_hbm, zrows_hbm, parts_out,
             src_v, dst_v, rows_v, acc, si, sg, ss):
    c = lax.axis_index("c")
    s = lax.axis_index("s")

    @pl.when(c == 0)
    def _():
        _chunked_row_copy(s, lambda r, n: pltpu.sync_copy(g_hbm.at[r], acc.at[r]))

    @pl.when(c != 0)
    def _():
        _chunked_row_copy(s, lambda r, n: pltpu.sync_copy(zrows_hbm.at[r], acc.at[r]))

    plsc.subcore_barrier()
    tile_base = (c * NS + s) * EDGES_PER_TILE

    def win_slice(w):
        return pl.ds(pl.multiple_of(tile_base + w * WIN, 8), WIN)

    def idx_start(w, b):
        pltpu.async_copy(src_hbm.at[win_slice(w)], src_v[b], si[b])
        pltpu.async_copy(dst_hbm.at[win_slice(w)], dst_v[b], si[b])

    def idx_wait(w, b):
        pltpu.make_async_copy(src_hbm.at[win_slice(w)], src_v[b], si[b]).wait()
        pltpu.make_async_copy(dst_hbm.at[win_slice(w)], dst_v[b], si[b]).wait()

    def gather_start(b):
        pltpu.async_copy(g_hbm.at[src_v[b]], rows_v[b], sg[b])

    def gather_wait(b):
        pltpu.make_async_copy(g_hbm.at[src_v[b]], rows_v[b], sg[b]).wait()

    def scatter_start(b):
        pltpu.async_copy(rows_v[b], acc.at[dst_v[b]], ss[b], add=True)

    def scatter_wait(b):
        pltpu.make_async_copy(rows_v[b], acc.at[dst_v[b]], ss[b]).wait()

    idx_start(0, 0)

    def body(i, carry):
        b = lax.rem(i, NBUF)
        bn = lax.rem(i + 1, NBUF)
        bp = lax.rem(i + NBUF - 1, NBUF)

        def at(bufsel, fn):
            for k in range(NBUF):
                pl.when(bufsel == k)(lambda kk=k: fn(kk))

        @pl.when(i >= 2)
        def _():
            at(bn, scatter_wait)

        @pl.when(i <= NWIN - 2)
        def _():
            for k in range(NBUF):
                pl.when(bn == k)(lambda kk=k: idx_start(i + 1, kk))

        @pl.when(i >= 1)
        def _():
            at(bp, gather_wait)
            at(bp, scatter_start)

        for k in range(NBUF):
            pl.when(b == k)(lambda kk=k: idx_wait(i, kk))
        at(b, gather_start)
        return carry

    lax.fori_loop(0, NWIN, body, 0)

    lb = (NWIN - 1) % NBUF
    gather_wait(lb)
    scatter_start(lb)
    scatter_wait((NWIN - 2) % NBUF)
    scatter_wait(lb)

    plsc.subcore_barrier()
    _chunked_row_copy(s, lambda r, n: pltpu.sync_copy(acc.at[r], parts_out.at[c, r]))


def _stage1_body(x_ref, w_ref, deg0_ref, deg1_ref, g_ref, dinv_ref):
    d = deg0_ref[...] + deg1_ref[...] + 1.0
    dinv = lax.rsqrt(d)
    h = jnp.dot(x_ref[...], w_ref[...], preferred_element_type=jnp.float32)
    g_ref[...] = h * dinv
    dinv_ref[...] = dinv


def _stage1(x, W1, deg0, deg1):
    return pl.pallas_call(
        _stage1_body,
        grid=(GRID,),
        in_specs=[
            pl.BlockSpec((BN, D), lambda i: (i, 0)),
            pl.BlockSpec((D, D), lambda i: (0, 0)),
            pl.BlockSpec((BN, 1), lambda i: (i, 0)),
            pl.BlockSpec((BN, 1), lambda i: (i, 0)),
        ],
        out_specs=[
            pl.BlockSpec((BN, D), lambda i: (i, 0)),
            pl.BlockSpec((BN, 1), lambda i: (i, 0)),
        ],
        out_shape=[
            jax.ShapeDtypeStruct((N, D), jnp.float32),
            jax.ShapeDtypeStruct((N, 1), jnp.float32),
        ],
    )(x, W1, deg0, deg1)


def _stage2_body(parts_ref, dinv_ref, w_ref, b_ref, g_ref):
    o = dinv_ref[...] * (parts_ref[0] + parts_ref[1]) + b_ref[...]
    h = jnp.dot(o, w_ref[...], preferred_element_type=jnp.float32)
    g_ref[...] = h * dinv_ref[...]


def _stage2(parts, dinv, W2, b1r):
    return pl.pallas_call(
        _stage2_body,
        grid=(GRID,),
        in_specs=[
            pl.BlockSpec((NC, BN, D), lambda i: (0, i, 0)),
            pl.BlockSpec((BN, 1), lambda i: (i, 0)),
            pl.BlockSpec((D, D), lambda i: (0, 0)),
            pl.BlockSpec((1, D), lambda i: (0, 0)),
        ],
        out_specs=pl.BlockSpec((BN, D), lambda i: (i, 0)),
        out_shape=jax.ShapeDtypeStruct((N, D), jnp.float32),
    )(parts, dinv, W2, b1r)


def _stage3_body(parts_ref, dinv_ref, b_ref, out_ref):
    o = dinv_ref[...] * (parts_ref[0] + parts_ref[1]) + b_ref[...]
    m = jnp.max(o, axis=1, keepdims=True)
    ex = jnp.exp(o - m)
    lse = jnp.log(jnp.sum(ex, axis=1, keepdims=True))
    out_ref[...] = o - m - lse


def _stage3(parts, dinv, b2r):
    return pl.pallas_call(
        _stage3_body,
        grid=(GRID,),
        in_specs=[
            pl.BlockSpec((NC, BN, D), lambda i: (0, i, 0)),
            pl.BlockSpec((BN, 1), lambda i: (i, 0)),
            pl.BlockSpec((1, D), lambda i: (0, 0)),
        ],
        out_specs=pl.BlockSpec((BN, D), lambda i: (i, 0)),
        out_shape=jax.ShapeDtypeStruct((N, D), jnp.float32),
    )(parts, dinv, b2r)


def kernel(x, edge_index, W1, b1, W2, b2, original_size):
    ones_win = jnp.ones((WIN,), jnp.float32)
    z1d = jnp.zeros((N,), jnp.float32)
    zrows = jnp.zeros((N, D), jnp.float32)
    src = edge_index[0]
    dst = edge_index[1]

    deg0, deg1 = _deg_sc(dst, ones_win, z1d)
    g1, dinv = _stage1(x, W1, jnp.reshape(deg0, (N, 1)), jnp.reshape(deg1, (N, 1)))
    parts1 = _spmm_sc(src, dst, g1, zrows)
    g2 = _stage2(parts1, dinv, W2, jnp.reshape(b1, (1, D)))
    parts2 = _spmm_sc(src, dst, g2, zrows)
    out = _stage3(parts2, dinv, jnp.reshape(b2, (1, D)))
    return out


# speedup vs baseline: 23.9664x; 1.0830x over previous
import functools

import jax
import jax.numpy as jnp
from jax import lax
from jax.experimental import pallas as pl
from jax.experimental.pallas import tpu as pltpu
from jax.experimental.pallas import tpu_sc as plsc

N = 10000
D = 128
E = 320000
NC = 2
NS = 16
EDGES_PER_TILE = E // (NC * NS)
WIN = 80
NWIN = EDGES_PER_TILE // WIN
CH = 624
LAST_START = CH * (NS - 1)
LAST = N - LAST_START
BN = 1000
GRID = N // BN

_mesh = plsc.VectorSubcoreMesh(
    core_axis_name="c", subcore_axis_name="s", num_cores=NC, num_subcores=NS
)


def _chunked_row_copy(s, copy_fn):
    @pl.when(s < NS - 1)
    def _():
        copy_fn(pl.ds(pl.multiple_of(s * CH, 8), CH), CH)

    @pl.when(s == NS - 1)
    def _():
        copy_fn(pl.ds(LAST_START, LAST), LAST)


@functools.partial(
    pl.kernel,
    out_type=[
        jax.ShapeDtypeStruct((N,), jnp.float32),
        jax.ShapeDtypeStruct((N,), jnp.float32),
    ],
    mesh=_mesh,
    scratch_types=[
        [pltpu.VMEM((WIN,), jnp.int32)] * 3,
        pltpu.VMEM((WIN,), jnp.float32),
        pltpu.VMEM((LAST,), jnp.float32),
        pltpu.VMEM_SHARED((N,), jnp.float32),
        [pltpu.SemaphoreType.DMA] * 3,
        [pltpu.SemaphoreType.DMA] * 3,
    ],
)
def _deg_sc(dst_hbm, ones_hbm, z1d_hbm, deg0_out, deg1_out, idx_v, ones_v, vbuf, acc, si, ss):
    c = lax.axis_index("c")
    s = lax.axis_index("s")

    def init_chunk(r, n):
        pltpu.sync_copy(z1d_hbm.at[r], vbuf.at[pl.ds(0, n)])
        pltpu.sync_copy(vbuf.at[pl.ds(0, n)], acc.at[r])

    _chunked_row_copy(s, init_chunk)
    pltpu.sync_copy(ones_hbm, ones_v)
    plsc.subcore_barrier()
    tile_base = (c * NS + s) * EDGES_PER_TILE

    def win_slice(w):
        return pl.ds(pl.multiple_of(tile_base + w * WIN, 8), WIN)

    def idx_start(w, b):
        pltpu.async_copy(dst_hbm.at[win_slice(w)], idx_v[b], si[b])

    def idx_wait(w, b):
        pltpu.make_async_copy(dst_hbm.at[win_slice(w)], idx_v[b], si[b]).wait()

    def scatter_start(b):
        pltpu.async_copy(ones_v, acc.at[idx_v[b]], ss[b], add=True)

    def scatter_wait(b):
        pltpu.make_async_copy(ones_v, acc.at[idx_v[b]], ss[b]).wait()

    idx_start(0, 0)

    def body(i, carry):
        b = lax.rem(i, 3)
        bn = lax.rem(i + 1, 3)

        @pl.when(i >= 2)
        def _():
            for k in range(3):
                pl.when(bn == k)(lambda kk=k: scatter_wait(kk))

        @pl.when(i <= NWIN - 2)
        def _():
            for k in range(3):
                pl.when(bn == k)(lambda kk=k: idx_start(i + 1, kk))

        for k in range(3):
            pl.when(b == k)(lambda kk=k: idx_wait(i, kk))
        for k in range(3):
            pl.when(b == k)(lambda kk=k: scatter_start(kk))
        return carry

    lax.fori_loop(0, NWIN, body, 0)
    scatter_wait((NWIN - 2) % 3)
    scatter_wait((NWIN - 1) % 3)
    plsc.subcore_barrier()

    def out_chunk(out_ref, r, n):
        pltpu.sync_copy(acc.at[r], vbuf.at[pl.ds(0, n)])
        pltpu.sync_copy(vbuf.at[pl.ds(0, n)], out_ref.at[r])

    @pl.when(c == 0)
    def _():
        _chunked_row_copy(s, lambda r, n: out_chunk(deg0_out, r, n))

    @pl.when(c != 0)
    def _():
        _chunked_row_copy(s, lambda r, n: out_chunk(deg1_out, r, n))


NBUF = 3


@functools.partial(
    pl.kernel,
    out_type=jax.ShapeDtypeStruct((NC, N, D), jnp.float32),
    mesh=_mesh,
    scratch_types=[
        [pltpu.VMEM((WIN,), jnp.int32)] * NBUF,
        [pltpu.VMEM((WIN,), jnp.int32)] * NBUF,
        [pltpu.VMEM((WIN, D), jnp.float32)] * NBUF,
        pltpu.VMEM_SHARED((N, D), jnp.float32),
        [pltpu.SemaphoreType.DMA] * NBUF,
        [pltpu.SemaphoreType.DMA] * NBUF,
        [pltpu.SemaphoreType.DMA] * NBUF,
    ],
)
def _spmm_sc(src_hbm, dst_hbm, g_hbm, zrows_hbm, parts_out,
             src_v, dst_v, rows_v, acc, si, sg, ss):
    c = lax.axis_index("c")
    s = lax.axis_index("s")

    @pl.when(c == 0)
    def _():
        _chunked_row_copy(s, lambda r, n: pltpu.sync_copy(g_hbm.at[r], acc.at[r]))

    @pl.when(c != 0)
    def _():
        _chunked_row_copy(s, lambda r, n: pltpu.sync_copy(zrows_hbm.at[r], acc.at[r]))

    plsc.subcore_barrier()
    tile_base = (c * NS + s) * EDGES_PER_TILE

    def win_slice(w):
        return pl.ds(pl.multiple_of(tile_base + w * WIN, 8), WIN)

    def idx_start(w, b):
        pltpu.async_copy(src_hbm.at[win_slice(w)], src_v[b], si[b])
        pltpu.async_copy(dst_hbm.at[win_slice(w)], dst_v[b], si[b])

    def idx_wait(w, b):
        pltpu.make_async_copy(src_hbm.at[win_slice(w)], src_v[b], si[b]).wait()
        pltpu.make_async_copy(dst_hbm.at[win_slice(w)], dst_v[b], si[b]).wait()

    def gather_start(b):
        pltpu.async_copy(g_hbm.at[src_v[b]], rows_v[b], sg[b])

    def gather_wait(b):
        pltpu.make_async_copy(g_hbm.at[src_v[b]], rows_v[b], sg[b]).wait()

    def scatter_start(b):
        pltpu.async_copy(rows_v[b], acc.at[dst_v[b]], ss[b], add=True)

    def scatter_wait(b):
        pltpu.make_async_copy(rows_v[b], acc.at[dst_v[b]], ss[b]).wait()

    idx_start(0, 0)

    def body(i, carry):
        b = lax.rem(i, NBUF)
        bn = lax.rem(i + 1, NBUF)
        bp = lax.rem(i + NBUF - 1, NBUF)

        def at(bufsel, fn):
            for k in range(NBUF):
                pl.when(bufsel == k)(lambda kk=k: fn(kk))

        @pl.when(i >= 2)
        def _():
            at(bn, scatter_wait)

        @pl.when(i <= NWIN - 2)
        def _():
            for k in range(NBUF):
                pl.when(bn == k)(lambda kk=k: idx_start(i + 1, kk))

        @pl.when(i >= 1)
        def _():
            at(bp, gather_wait)
            at(bp, scatter_start)

        for k in range(NBUF):
            pl.when(b == k)(lambda kk=k: idx_wait(i, kk))
        at(b, gather_start)
        return carry

    lax.fori_loop(0, NWIN, body, 0)

    lb = (NWIN - 1) % NBUF
    gather_wait(lb)
    scatter_start(lb)
    scatter_wait((NWIN - 2) % NBUF)
    scatter_wait(lb)

    plsc.subcore_barrier()
    _chunked_row_copy(s, lambda r, n: pltpu.sync_copy(acc.at[r], parts_out.at[c, r]))


def _stage1_body(x_ref, w_ref, deg0_ref, deg1_ref, g_ref, dinv_ref):
    d = deg0_ref[...] + deg1_ref[...] + 1.0
    dinv = lax.rsqrt(d)
    h = jnp.dot(x_ref[...], w_ref[...], preferred_element_type=jnp.float32)
    g_ref[...] = h * dinv
    dinv_ref[...] = dinv


def _stage1(x, W1, deg0, deg1):
    return pl.pallas_call(
        _stage1_body,
        grid=(GRID,),
        in_specs=[
            pl.BlockSpec((BN, D), lambda i: (i, 0)),
            pl.BlockSpec((D, D), lambda i: (0, 0)),
            pl.BlockSpec((BN, 1), lambda i: (i, 0)),
            pl.BlockSpec((BN, 1), lambda i: (i, 0)),
        ],
        out_specs=[
            pl.BlockSpec((BN, D), lambda i: (i, 0)),
            pl.BlockSpec((BN, 1), lambda i: (i, 0)),
        ],
        out_shape=[
            jax.ShapeDtypeStruct((N, D), jnp.float32),
            jax.ShapeDtypeStruct((N, 1), jnp.float32),
        ],
    )(x, W1, deg0, deg1)


def _stage2_body(parts_ref, dinv_ref, w_ref, b_ref, g_ref):
    o = dinv_ref[...] * (parts_ref[0] + parts_ref[1]) + b_ref[...]
    h = jnp.dot(o, w_ref[...], preferred_element_type=jnp.float32)
    g_ref[...] = h * dinv_ref[...]


def _stage2(parts, dinv, W2, b1r):
    return pl.pallas_call(
        _stage2_body,
        grid=(GRID,),
        in_specs=[
            pl.BlockSpec((NC, BN, D), lambda i: (0, i, 0)),
            pl.BlockSpec((BN, 1), lambda i: (i, 0)),
            pl.BlockSpec((D, D), lambda i: (0, 0)),
            pl.BlockSpec((1, D), lambda i: (0, 0)),
        ],
        out_specs=pl.BlockSpec((BN, D), lambda i: (i, 0)),
        out_shape=jax.ShapeDtypeStruct((N, D), jnp.float32),
    )(parts, dinv, W2, b1r)


def _stage3_body(parts_ref, dinv_ref, b_ref, out_ref):
    o = dinv_ref[...] * (parts_ref[0] + parts_ref[1]) + b_ref[...]
    m = jnp.max(o, axis=1, keepdims=True)
    ex = jnp.exp(o - m)
    lse = jnp.log(jnp.sum(ex, axis=1, keepdims=True))
    out_ref[...] = o - m - lse


def _stage3(parts, dinv, b2r):
    return pl.pallas_call(
        _stage3_body,
        grid=(GRID,),
        in_specs=[
            pl.BlockSpec((NC, BN, D), lambda i: (0, i, 0)),
            pl.BlockSpec((BN, 1), lambda i: (i, 0)),
            pl.BlockSpec((1, D), lambda i: (0, 0)),
        ],
        out_specs=pl.BlockSpec((BN, D), lambda i: (i, 0)),
        out_shape=jax.ShapeDtypeStruct((N, D), jnp.float32),
    )(parts, dinv, b2r)


def kernel(x, edge_index, W1, b1, W2, b2, original_size):
    ones_win = jnp.ones((WIN,), jnp.float32)
    z1d = jnp.zeros((N,), jnp.float32)
    zrows = jnp.zeros((N, D), jnp.float32)
    src = edge_index[0]
    dst = edge_index[1]

    deg0, deg1 = _deg_sc(dst, ones_win, z1d)
    g1, dinv = _stage1(x, W1, jnp.reshape(deg0, (N, 1)), jnp.reshape(deg1, (N, 1)))
    parts1 = _spmm_sc(src, dst, g1, zrows)
    g2 = _stage2(parts1, dinv, W2, jnp.reshape(b1, (1, D)))
    parts2 = _spmm_sc(src, dst, g2, zrows)
    out = _stage3(parts2, dinv, jnp.reshape(b2, (1, D)))
    return out


# speedup vs baseline: 25.5346x; 1.0654x over previous
import functools

import jax
import jax.numpy as jnp
from jax import lax
from jax.experimental import pallas as pl
from jax.experimental.pallas import tpu as pltpu
from jax.experimental.pallas import tpu_sc as plsc

N = 10000
D = 128
E = 320000
NC = 2
NS = 16
EDGES_PER_TILE = E // (NC * NS)
WIN = 96
NWIN = EDGES_PER_TILE // WIN
TAIL = EDGES_PER_TILE - NWIN * WIN
TAIL_OFF = NWIN * WIN
CH = 624
LAST_START = CH * (NS - 1)
LAST = N - LAST_START
BN = 1000
GRID = N // BN

_mesh = plsc.VectorSubcoreMesh(
    core_axis_name="c", subcore_axis_name="s", num_cores=NC, num_subcores=NS
)


def _chunked_row_copy(s, copy_fn):
    @pl.when(s < NS - 1)
    def _():
        copy_fn(pl.ds(pl.multiple_of(s * CH, 8), CH), CH)

    @pl.when(s == NS - 1)
    def _():
        copy_fn(pl.ds(LAST_START, LAST), LAST)


@functools.partial(
    pl.kernel,
    out_type=[
        jax.ShapeDtypeStruct((N,), jnp.float32),
        jax.ShapeDtypeStruct((N,), jnp.float32),
    ],
    mesh=_mesh,
    scratch_types=[
        [pltpu.VMEM((WIN,), jnp.int32)] * 3,
        pltpu.VMEM((WIN,), jnp.float32),
        pltpu.VMEM((TAIL,), jnp.int32),
        pltpu.VMEM((TAIL,), jnp.float32),
        pltpu.VMEM((LAST,), jnp.float32),
        pltpu.VMEM_SHARED((N,), jnp.float32),
        [pltpu.SemaphoreType.DMA] * 3,
        [pltpu.SemaphoreType.DMA] * 3,
    ],
)
def _deg_sc(dst_hbm, ones_hbm, z1d_hbm, deg0_out, deg1_out,
            idx_v, ones_v, idx_t, ones_t, vbuf, acc, si, ss):
    c = lax.axis_index("c")
    s = lax.axis_index("s")

    def init_chunk(r, n):
        pltpu.sync_copy(z1d_hbm.at[r], vbuf.at[pl.ds(0, n)])
        pltpu.sync_copy(vbuf.at[pl.ds(0, n)], acc.at[r])

    _chunked_row_copy(s, init_chunk)
    pltpu.sync_copy(ones_hbm, ones_v)
    plsc.subcore_barrier()
    tile_base = (c * NS + s) * EDGES_PER_TILE

    def win_slice(w):
        return pl.ds(pl.multiple_of(tile_base + w * WIN, 8), WIN)

    def idx_start(w, b):
        pltpu.async_copy(dst_hbm.at[win_slice(w)], idx_v[b], si[b])

    def idx_wait(w, b):
        pltpu.make_async_copy(dst_hbm.at[win_slice(w)], idx_v[b], si[b]).wait()

    def scatter_start(b):
        pltpu.async_copy(ones_v, acc.at[idx_v[b]], ss[b], add=True)

    def scatter_wait(b):
        pltpu.make_async_copy(ones_v, acc.at[idx_v[b]], ss[b]).wait()

    idx_start(0, 0)

    def body(i, carry):
        b = lax.rem(i, 3)
        bn = lax.rem(i + 1, 3)

        @pl.when(i >= 2)
        def _():
            for k in range(3):
                pl.when(bn == k)(lambda kk=k: scatter_wait(kk))

        @pl.when(i <= NWIN - 2)
        def _():
            for k in range(3):
                pl.when(bn == k)(lambda kk=k: idx_start(i + 1, kk))

        for k in range(3):
            pl.when(b == k)(lambda kk=k: idx_wait(i, kk))
        for k in range(3):
            pl.when(b == k)(lambda kk=k: scatter_start(kk))
        return carry

    lax.fori_loop(0, NWIN, body, 0)
    scatter_wait((NWIN - 2) % 3)
    scatter_wait((NWIN - 1) % 3)

    pltpu.sync_copy(ones_hbm.at[pl.ds(0, TAIL)], ones_t)
    pltpu.sync_copy(dst_hbm.at[pl.ds(pl.multiple_of(tile_base + TAIL_OFF, 8), TAIL)], idx_t)
    pltpu.sync_copy(ones_t, acc.at[idx_t], add=True)
    plsc.subcore_barrier()

    def out_chunk(out_ref, r, n):
        pltpu.sync_copy(acc.at[r], vbuf.at[pl.ds(0, n)])
        pltpu.sync_copy(vbuf.at[pl.ds(0, n)], out_ref.at[r])

    @pl.when(c == 0)
    def _():
        _chunked_row_copy(s, lambda r, n: out_chunk(deg0_out, r, n))

    @pl.when(c != 0)
    def _():
        _chunked_row_copy(s, lambda r, n: out_chunk(deg1_out, r, n))


NBUF = 3


@functools.partial(
    pl.kernel,
    out_type=jax.ShapeDtypeStruct((NC, N, D), jnp.float32),
    mesh=_mesh,
    scratch_types=[
        [pltpu.VMEM((WIN,), jnp.int32)] * NBUF,
        [pltpu.VMEM((WIN,), jnp.int32)] * NBUF,
        [pltpu.VMEM((WIN, D), jnp.float32)] * NBUF,
        pltpu.VMEM((TAIL,), jnp.int32),
        pltpu.VMEM((TAIL,), jnp.int32),
        pltpu.VMEM((TAIL, D), jnp.float32),
        pltpu.VMEM_SHARED((N, D), jnp.float32),
        [pltpu.SemaphoreType.DMA] * NBUF,
        [pltpu.SemaphoreType.DMA] * NBUF,
        [pltpu.SemaphoreType.DMA] * NBUF,
    ],
)
def _spmm_sc(src_hbm, dst_hbm, g_hbm, zrows_hbm, parts_out,
             src_v, dst_v, rows_v, src_t, dst_t, rows_t, acc, si, sg, ss):
    c = lax.axis_index("c")
    s = lax.axis_index("s")

    @pl.when(c == 0)
    def _():
        _chunked_row_copy(s, lambda r, n: pltpu.sync_copy(g_hbm.at[r], acc.at[r]))

    @pl.when(c != 0)
    def _():
        _chunked_row_copy(s, lambda r, n: pltpu.sync_copy(zrows_hbm.at[r], acc.at[r]))

    plsc.subcore_barrier()
    tile_base = (c * NS + s) * EDGES_PER_TILE

    def win_slice(w):
        return pl.ds(pl.multiple_of(tile_base + w * WIN, 8), WIN)

    def idx_start(w, b):
        pltpu.async_copy(src_hbm.at[win_slice(w)], src_v[b], si[b])
        pltpu.async_copy(dst_hbm.at[win_slice(w)], dst_v[b], si[b])

    def idx_wait(w, b):
        pltpu.make_async_copy(src_hbm.at[win_slice(w)], src_v[b], si[b]).wait()
        pltpu.make_async_copy(dst_hbm.at[win_slice(w)], dst_v[b], si[b]).wait()

    def gather_start(b):
        pltpu.async_copy(g_hbm.at[src_v[b]], rows_v[b], sg[b])

    def gather_wait(b):
        pltpu.make_async_copy(g_hbm.at[src_v[b]], rows_v[b], sg[b]).wait()

    def scatter_start(b):
        pltpu.async_copy(rows_v[b], acc.at[dst_v[b]], ss[b], add=True)

    def scatter_wait(b):
        pltpu.make_async_copy(rows_v[b], acc.at[dst_v[b]], ss[b]).wait()

    idx_start(0, 0)

    def body(i, carry):
        b = lax.rem(i, NBUF)
        bn = lax.rem(i + 1, NBUF)
        bp = lax.rem(i + NBUF - 1, NBUF)

        def at(bufsel, fn):
            for k in range(NBUF):
                pl.when(bufsel == k)(lambda kk=k: fn(kk))

        @pl.when(i >= 2)
        def _():
            at(bn, scatter_wait)

        @pl.when(i <= NWIN - 2)
        def _():
            for k in range(NBUF):
                pl.when(bn == k)(lambda kk=k: idx_start(i + 1, kk))

        @pl.when(i >= 1)
        def _():
            at(bp, gather_wait)
            at(bp, scatter_start)

        for k in range(NBUF):
            pl.when(b == k)(lambda kk=k: idx_wait(i, kk))
        at(b, gather_start)
        return carry

    lax.fori_loop(0, NWIN, body, 0)

    lb = (NWIN - 1) % NBUF
    gather_wait(lb)
    scatter_start(lb)
    scatter_wait((NWIN - 2) % NBUF)
    scatter_wait(lb)

    tsl = pl.ds(pl.multiple_of(tile_base + TAIL_OFF, 8), TAIL)
    pltpu.sync_copy(src_hbm.at[tsl], src_t)
    pltpu.sync_copy(dst_hbm.at[tsl], dst_t)
    pltpu.async_copy(g_hbm.at[src_t], rows_t, si[0]).wait()
    pltpu.sync_copy(rows_t, acc.at[dst_t], add=True)

    plsc.subcore_barrier()
    _chunked_row_copy(s, lambda r, n: pltpu.sync_copy(acc.at[r], parts_out.at[c, r]))


def _stage1_body(x_ref, w_ref, deg0_ref, deg1_ref, g_ref, dinv_ref):
    d = deg0_ref[...] + deg1_ref[...] + 1.0
    dinv = lax.rsqrt(d)
    h = jnp.dot(x_ref[...], w_ref[...], preferred_element_type=jnp.float32)
    g_ref[...] = h * dinv
    dinv_ref[...] = dinv


def _stage1(x, W1, deg0, deg1):
    return pl.pallas_call(
        _stage1_body,
        grid=(GRID,),
        in_specs=[
            pl.BlockSpec((BN, D), lambda i: (i, 0)),
            pl.BlockSpec((D, D), lambda i: (0, 0)),
            pl.BlockSpec((BN, 1), lambda i: (i, 0)),
            pl.BlockSpec((BN, 1), lambda i: (i, 0)),
        ],
        out_specs=[
            pl.BlockSpec((BN, D), lambda i: (i, 0)),
            pl.BlockSpec((BN, 1), lambda i: (i, 0)),
        ],
        out_shape=[
            jax.ShapeDtypeStruct((N, D), jnp.float32),
            jax.ShapeDtypeStruct((N, 1), jnp.float32),
        ],
    )(x, W1, deg0, deg1)


def _stage2_body(parts_ref, dinv_ref, w_ref, b_ref, g_ref):
    o = dinv_ref[...] * (parts_ref[0] + parts_ref[1]) + b_ref[...]
    h = jnp.dot(o, w_ref[...], preferred_element_type=jnp.float32)
    g_ref[...] = h * dinv_ref[...]


def _stage2(parts, dinv, W2, b1r):
    return pl.pallas_call(
        _stage2_body,
        grid=(GRID,),
        in_specs=[
            pl.BlockSpec((NC, BN, D), lambda i: (0, i, 0)),
            pl.BlockSpec((BN, 1), lambda i: (i, 0)),
            pl.BlockSpec((D, D), lambda i: (0, 0)),
            pl.BlockSpec((1, D), lambda i: (0, 0)),
        ],
        out_specs=pl.BlockSpec((BN, D), lambda i: (i, 0)),
        out_shape=jax.ShapeDtypeStruct((N, D), jnp.float32),
    )(parts, dinv, W2, b1r)


def _stage3_body(parts_ref, dinv_ref, b_ref, out_ref):
    o = dinv_ref[...] * (parts_ref[0] + parts_ref[1]) + b_ref[...]
    m = jnp.max(o, axis=1, keepdims=True)
    ex = jnp.exp(o - m)
    lse = jnp.log(jnp.sum(ex, axis=1, keepdims=True))
    out_ref[...] = o - m - lse


def _stage3(parts, dinv, b2r):
    return pl.pallas_call(
        _stage3_body,
        grid=(GRID,),
        in_specs=[
            pl.BlockSpec((NC, BN, D), lambda i: (0, i, 0)),
            pl.BlockSpec((BN, 1), lambda i: (i, 0)),
            pl.BlockSpec((1, D), lambda i: (0, 0)),
        ],
        out_specs=pl.BlockSpec((BN, D), lambda i: (i, 0)),
        out_shape=jax.ShapeDtypeStruct((N, D), jnp.float32),
    )(parts, dinv, b2r)


def kernel(x, edge_index, W1, b1, W2, b2, original_size):
    ones_win = jnp.ones((WIN,), jnp.float32)
    z1d = jnp.zeros((N,), jnp.float32)
    zrows = jnp.zeros((N, D), jnp.float32)
    src = edge_index[0]
    dst = edge_index[1]

    deg0, deg1 = _deg_sc(dst, ones_win, z1d)
    g1, dinv = _stage1(x, W1, jnp.reshape(deg0, (N, 1)), jnp.reshape(deg1, (N, 1)))
    parts1 = _spmm_sc(src, dst, g1, zrows)
    g2 = _stage2(parts1, dinv, W2, jnp.reshape(b1, (1, D)))
    parts2 = _spmm_sc(src, dst, g2, zrows)
    out = _stage3(parts2, dinv, jnp.reshape(b2, (1, D)))
    return out


# speedup vs baseline: 26.2337x; 1.0274x over previous
import functools

import jax
import jax.numpy as jnp
from jax import lax
from jax.experimental import pallas as pl
from jax.experimental.pallas import tpu as pltpu
from jax.experimental.pallas import tpu_sc as plsc

N = 10000
D = 128
E = 320000
NC = 2
NS = 16
EDGES_PER_TILE = E // (NC * NS)
WIN = 104
NWIN = EDGES_PER_TILE // WIN
TAIL = EDGES_PER_TILE - NWIN * WIN
TAIL_OFF = NWIN * WIN
CH = 624
LAST_START = CH * (NS - 1)
LAST = N - LAST_START
BN = 1000
GRID = N // BN

_mesh = plsc.VectorSubcoreMesh(
    core_axis_name="c", subcore_axis_name="s", num_cores=NC, num_subcores=NS
)


def _chunked_row_copy(s, copy_fn):
    @pl.when(s < NS - 1)
    def _():
        copy_fn(pl.ds(pl.multiple_of(s * CH, 8), CH), CH)

    @pl.when(s == NS - 1)
    def _():
        copy_fn(pl.ds(LAST_START, LAST), LAST)


@functools.partial(
    pl.kernel,
    out_type=[
        jax.ShapeDtypeStruct((N,), jnp.float32),
        jax.ShapeDtypeStruct((N,), jnp.float32),
    ],
    mesh=_mesh,
    scratch_types=[
        [pltpu.VMEM((WIN,), jnp.int32)] * 3,
        pltpu.VMEM((WIN,), jnp.float32),
        pltpu.VMEM((TAIL,), jnp.int32),
        pltpu.VMEM((TAIL,), jnp.float32),
        pltpu.VMEM((LAST,), jnp.float32),
        pltpu.VMEM_SHARED((N,), jnp.float32),
        [pltpu.SemaphoreType.DMA] * 3,
        [pltpu.SemaphoreType.DMA] * 3,
    ],
)
def _deg_sc(dst_hbm, ones_hbm, z1d_hbm, deg0_out, deg1_out,
            idx_v, ones_v, idx_t, ones_t, vbuf, acc, si, ss):
    c = lax.axis_index("c")
    s = lax.axis_index("s")

    def init_chunk(r, n):
        pltpu.sync_copy(z1d_hbm.at[r], vbuf.at[pl.ds(0, n)])
        pltpu.sync_copy(vbuf.at[pl.ds(0, n)], acc.at[r])

    _chunked_row_copy(s, init_chunk)
    pltpu.sync_copy(ones_hbm, ones_v)
    plsc.subcore_barrier()
    tile_base = (c * NS + s) * EDGES_PER_TILE

    def win_slice(w):
        return pl.ds(pl.multiple_of(tile_base + w * WIN, 8), WIN)

    def idx_start(w, b):
        pltpu.async_copy(dst_hbm.at[win_slice(w)], idx_v[b], si[b])

    def idx_wait(w, b):
        pltpu.make_async_copy(dst_hbm.at[win_slice(w)], idx_v[b], si[b]).wait()

    def scatter_start(b):
        pltpu.async_copy(ones_v, acc.at[idx_v[b]], ss[b], add=True)

    def scatter_wait(b):
        pltpu.make_async_copy(ones_v, acc.at[idx_v[b]], ss[b]).wait()

    idx_start(0, 0)

    def body(i, carry):
        b = lax.rem(i, 3)
        bn = lax.rem(i + 1, 3)

        @pl.when(i >= 2)
        def _():
            for k in range(3):
                pl.when(bn == k)(lambda kk=k: scatter_wait(kk))

        @pl.when(i <= NWIN - 2)
        def _():
            for k in range(3):
                pl.when(bn == k)(lambda kk=k: idx_start(i + 1, kk))

        for k in range(3):
            pl.when(b == k)(lambda kk=k: idx_wait(i, kk))
        for k in range(3):
            pl.when(b == k)(lambda kk=k: scatter_start(kk))
        return carry

    lax.fori_loop(0, NWIN, body, 0)
    scatter_wait((NWIN - 2) % 3)
    scatter_wait((NWIN - 1) % 3)

    pltpu.sync_copy(ones_hbm.at[pl.ds(0, TAIL)], ones_t)
    pltpu.sync_copy(dst_hbm.at[pl.ds(pl.multiple_of(tile_base + TAIL_OFF, 8), TAIL)], idx_t)
    pltpu.sync_copy(ones_t, acc.at[idx_t], add=True)
    plsc.subcore_barrier()

    def out_chunk(out_ref, r, n):
        pltpu.sync_copy(acc.at[r], vbuf.at[pl.ds(0, n)])
        pltpu.sync_copy(vbuf.at[pl.ds(0, n)], out_ref.at[r])

    @pl.when(c == 0)
    def _():
        _chunked_row_copy(s, lambda r, n: out_chunk(deg0_out, r, n))

    @pl.when(c != 0)
    def _():
        _chunked_row_copy(s, lambda r, n: out_chunk(deg1_out, r, n))


NBUF = 3


@functools.partial(
    pl.kernel,
    out_type=jax.ShapeDtypeStruct((NC, N, D), jnp.float32),
    mesh=_mesh,
    scratch_types=[
        [pltpu.VMEM((WIN,), jnp.int32)] * NBUF,
        [pltpu.VMEM((WIN,), jnp.int32)] * NBUF,
        [pltpu.VMEM((WIN, D), jnp.float32)] * NBUF,
        pltpu.VMEM((TAIL,), jnp.int32),
        pltpu.VMEM((TAIL,), jnp.int32),
        pltpu.VMEM((TAIL, D), jnp.float32),
        pltpu.VMEM_SHARED((N, D), jnp.float32),
        [pltpu.SemaphoreType.DMA] * NBUF,
        [pltpu.SemaphoreType.DMA] * NBUF,
        [pltpu.SemaphoreType.DMA] * NBUF,
    ],
)
def _spmm_sc(src_hbm, dst_hbm, g_hbm, zrows_hbm, parts_out,
             src_v, dst_v, rows_v, src_t, dst_t, rows_t, acc, si, sg, ss):
    c = lax.axis_index("c")
    s = lax.axis_index("s")

    @pl.when(c == 0)
    def _():
        _chunked_row_copy(s, lambda r, n: pltpu.sync_copy(g_hbm.at[r], acc.at[r]))

    @pl.when(c != 0)
    def _():
        _chunked_row_copy(s, lambda r, n: pltpu.sync_copy(zrows_hbm.at[r], acc.at[r]))

    plsc.subcore_barrier()
    tile_base = (c * NS + s) * EDGES_PER_TILE

    def win_slice(w):
        return pl.ds(pl.multiple_of(tile_base + w * WIN, 8), WIN)

    def idx_start(w, b):
        pltpu.async_copy(src_hbm.at[win_slice(w)], src_v[b], si[b])
        pltpu.async_copy(dst_hbm.at[win_slice(w)], dst_v[b], si[b])

    def idx_wait(w, b):
        pltpu.make_async_copy(src_hbm.at[win_slice(w)], src_v[b], si[b]).wait()
        pltpu.make_async_copy(dst_hbm.at[win_slice(w)], dst_v[b], si[b]).wait()

    def gather_start(b):
        pltpu.async_copy(g_hbm.at[src_v[b]], rows_v[b], sg[b])

    def gather_wait(b):
        pltpu.make_async_copy(g_hbm.at[src_v[b]], rows_v[b], sg[b]).wait()

    def scatter_start(b):
        pltpu.async_copy(rows_v[b], acc.at[dst_v[b]], ss[b], add=True)

    def scatter_wait(b):
        pltpu.make_async_copy(rows_v[b], acc.at[dst_v[b]], ss[b]).wait()

    idx_start(0, 0)

    def body(i, carry):
        b = lax.rem(i, NBUF)
        bn = lax.rem(i + 1, NBUF)
        bp = lax.rem(i + NBUF - 1, NBUF)

        def at(bufsel, fn):
            for k in range(NBUF):
                pl.when(bufsel == k)(lambda kk=k: fn(kk))

        @pl.when(i >= NBUF - 1)
        def _():
            at(bn, scatter_wait)

        @pl.when(i <= NWIN - 2)
        def _():
            for k in range(NBUF):
                pl.when(bn == k)(lambda kk=k: idx_start(i + 1, kk))

        @pl.when(i >= 1)
        def _():
            at(bp, gather_wait)
            at(bp, scatter_start)

        for k in range(NBUF):
            pl.when(b == k)(lambda kk=k: idx_wait(i, kk))
        at(b, gather_start)
        return carry

    lax.fori_loop(0, NWIN, body, 0)

    lb = (NWIN - 1) % NBUF
    gather_wait(lb)
    scatter_start(lb)
    for j in range(max(NWIN - NBUF + 1, 0), NWIN):
        scatter_wait(j % NBUF)

    tsl = pl.ds(pl.multiple_of(tile_base + TAIL_OFF, 8), TAIL)
    pltpu.sync_copy(src_hbm.at[tsl], src_t)
    pltpu.sync_copy(dst_hbm.at[tsl], dst_t)
    pltpu.async_copy(g_hbm.at[src_t], rows_t, si[0]).wait()
    pltpu.sync_copy(rows_t, acc.at[dst_t], add=True)

    plsc.subcore_barrier()
    _chunked_row_copy(s, lambda r, n: pltpu.sync_copy(acc.at[r], parts_out.at[c, r]))


def _stage1_body(x_ref, w_ref, deg0_ref, deg1_ref, g_ref, dinv_ref):
    d = deg0_ref[...] + deg1_ref[...] + 1.0
    dinv = lax.rsqrt(d)
    h = jnp.dot(x_ref[...], w_ref[...], preferred_element_type=jnp.float32)
    g_ref[...] = h * dinv
    dinv_ref[...] = dinv


def _stage1(x, W1, deg0, deg1):
    return pl.pallas_call(
        _stage1_body,
        grid=(GRID,),
        in_specs=[
            pl.BlockSpec((BN, D), lambda i: (i, 0)),
            pl.BlockSpec((D, D), lambda i: (0, 0)),
            pl.BlockSpec((BN, 1), lambda i: (i, 0)),
            pl.BlockSpec((BN, 1), lambda i: (i, 0)),
        ],
        out_specs=[
            pl.BlockSpec((BN, D), lambda i: (i, 0)),
            pl.BlockSpec((BN, 1), lambda i: (i, 0)),
        ],
        out_shape=[
            jax.ShapeDtypeStruct((N, D), jnp.float32),
            jax.ShapeDtypeStruct((N, 1), jnp.float32),
        ],
    )(x, W1, deg0, deg1)


def _stage2_body(parts_ref, dinv_ref, w_ref, b_ref, g_ref):
    o = dinv_ref[...] * (parts_ref[0] + parts_ref[1]) + b_ref[...]
    h = jnp.dot(o, w_ref[...], preferred_element_type=jnp.float32)
    g_ref[...] = h * dinv_ref[...]


def _stage2(parts, dinv, W2, b1r):
    return pl.pallas_call(
        _stage2_body,
        grid=(GRID,),
        in_specs=[
            pl.BlockSpec((NC, BN, D), lambda i: (0, i, 0)),
            pl.BlockSpec((BN, 1), lambda i: (i, 0)),
            pl.BlockSpec((D, D), lambda i: (0, 0)),
            pl.BlockSpec((1, D), lambda i: (0, 0)),
        ],
        out_specs=pl.BlockSpec((BN, D), lambda i: (i, 0)),
        out_shape=jax.ShapeDtypeStruct((N, D), jnp.float32),
    )(parts, dinv, W2, b1r)


def _stage3_body(parts_ref, dinv_ref, b_ref, out_ref):
    o = dinv_ref[...] * (parts_ref[0] + parts_ref[1]) + b_ref[...]
    m = jnp.max(o, axis=1, keepdims=True)
    ex = jnp.exp(o - m)
    lse = jnp.log(jnp.sum(ex, axis=1, keepdims=True))
    out_ref[...] = o - m - lse


def _stage3(parts, dinv, b2r):
    return pl.pallas_call(
        _stage3_body,
        grid=(GRID,),
        in_specs=[
            pl.BlockSpec((NC, BN, D), lambda i: (0, i, 0)),
            pl.BlockSpec((BN, 1), lambda i: (i, 0)),
            pl.BlockSpec((1, D), lambda i: (0, 0)),
        ],
        out_specs=pl.BlockSpec((BN, D), lambda i: (i, 0)),
        out_shape=jax.ShapeDtypeStruct((N, D), jnp.float32),
    )(parts, dinv, b2r)


def kernel(x, edge_index, W1, b1, W2, b2, original_size):
    ones_win = jnp.ones((WIN,), jnp.float32)
    z1d = jnp.zeros((N,), jnp.float32)
    zrows = jnp.zeros((N, D), jnp.float32)
    src = edge_index[0]
    dst = edge_index[1]

    deg0, deg1 = _deg_sc(dst, ones_win, z1d)
    g1, dinv = _stage1(x, W1, jnp.reshape(deg0, (N, 1)), jnp.reshape(deg1, (N, 1)))
    parts1 = _spmm_sc(src, dst, g1, zrows)
    g2 = _stage2(parts1, dinv, W2, jnp.reshape(b1, (1, D)))
    parts2 = _spmm_sc(src, dst, g2, zrows)
    out = _stage3(parts2, dinv, jnp.reshape(b2, (1, D)))
    return out


# speedup vs baseline: 26.7059x; 1.0180x over previous
import functools

import jax
import jax.numpy as jnp
from jax import lax
from jax.experimental import pallas as pl
from jax.experimental.pallas import tpu as pltpu
from jax.experimental.pallas import tpu_sc as plsc

N = 10000
D = 128
E = 320000
NC = 2
NS = 16
EDGES_PER_TILE = E // (NC * NS)
WIN = 104
NWIN = EDGES_PER_TILE // WIN
TAIL = EDGES_PER_TILE - NWIN * WIN
TAIL_OFF = NWIN * WIN
CH = 624
LAST_START = CH * (NS - 1)
LAST = N - LAST_START
BN = 2000
GRID = N // BN

_mesh = plsc.VectorSubcoreMesh(
    core_axis_name="c", subcore_axis_name="s", num_cores=NC, num_subcores=NS
)


def _chunked_row_copy(s, copy_fn):
    @pl.when(s < NS - 1)
    def _():
        copy_fn(pl.ds(pl.multiple_of(s * CH, 8), CH), CH)

    @pl.when(s == NS - 1)
    def _():
        copy_fn(pl.ds(LAST_START, LAST), LAST)


@functools.partial(
    pl.kernel,
    out_type=[
        jax.ShapeDtypeStruct((N,), jnp.float32),
        jax.ShapeDtypeStruct((N,), jnp.float32),
    ],
    mesh=_mesh,
    scratch_types=[
        [pltpu.VMEM((WIN,), jnp.int32)] * 3,
        pltpu.VMEM((WIN,), jnp.float32),
        pltpu.VMEM((TAIL,), jnp.int32),
        pltpu.VMEM((TAIL,), jnp.float32),
        pltpu.VMEM((LAST,), jnp.float32),
        pltpu.VMEM_SHARED((N,), jnp.float32),
        [pltpu.SemaphoreType.DMA] * 3,
        [pltpu.SemaphoreType.DMA] * 3,
    ],
)
def _deg_sc(dst_hbm, ones_hbm, z1d_hbm, deg0_out, deg1_out,
            idx_v, ones_v, idx_t, ones_t, vbuf, acc, si, ss):
    c = lax.axis_index("c")
    s = lax.axis_index("s")

    def init_chunk(r, n):
        pltpu.sync_copy(z1d_hbm.at[r], vbuf.at[pl.ds(0, n)])
        pltpu.sync_copy(vbuf.at[pl.ds(0, n)], acc.at[r])

    _chunked_row_copy(s, init_chunk)
    pltpu.sync_copy(ones_hbm, ones_v)
    plsc.subcore_barrier()
    tile_base = (c * NS + s) * EDGES_PER_TILE

    def win_slice(w):
        return pl.ds(pl.multiple_of(tile_base + w * WIN, 8), WIN)

    def idx_start(w, b):
        pltpu.async_copy(dst_hbm.at[win_slice(w)], idx_v[b], si[b])

    def idx_wait(w, b):
        pltpu.make_async_copy(dst_hbm.at[win_slice(w)], idx_v[b], si[b]).wait()

    def scatter_start(b):
        pltpu.async_copy(ones_v, acc.at[idx_v[b]], ss[b], add=True)

    def scatter_wait(b):
        pltpu.make_async_copy(ones_v, acc.at[idx_v[b]], ss[b]).wait()

    idx_start(0, 0)

    def body(i, carry):
        b = lax.rem(i, 3)
        bn = lax.rem(i + 1, 3)

        @pl.when(i >= 2)
        def _():
            for k in range(3):
                pl.when(bn == k)(lambda kk=k: scatter_wait(kk))

        @pl.when(i <= NWIN - 2)
        def _():
            for k in range(3):
                pl.when(bn == k)(lambda kk=k: idx_start(i + 1, kk))

        for k in range(3):
            pl.when(b == k)(lambda kk=k: idx_wait(i, kk))
        for k in range(3):
            pl.when(b == k)(lambda kk=k: scatter_start(kk))
        return carry

    lax.fori_loop(0, NWIN, body, 0)
    scatter_wait((NWIN - 2) % 3)
    scatter_wait((NWIN - 1) % 3)

    pltpu.sync_copy(ones_hbm.at[pl.ds(0, TAIL)], ones_t)
    pltpu.sync_copy(dst_hbm.at[pl.ds(pl.multiple_of(tile_base + TAIL_OFF, 8), TAIL)], idx_t)
    pltpu.sync_copy(ones_t, acc.at[idx_t], add=True)
    plsc.subcore_barrier()

    def out_chunk(out_ref, r, n):
        pltpu.sync_copy(acc.at[r], vbuf.at[pl.ds(0, n)])
        pltpu.sync_copy(vbuf.at[pl.ds(0, n)], out_ref.at[r])

    @pl.when(c == 0)
    def _():
        _chunked_row_copy(s, lambda r, n: out_chunk(deg0_out, r, n))

    @pl.when(c != 0)
    def _():
        _chunked_row_copy(s, lambda r, n: out_chunk(deg1_out, r, n))


NBUF = 3


@functools.partial(
    pl.kernel,
    out_type=jax.ShapeDtypeStruct((NC, N, D), jnp.float32),
    mesh=_mesh,
    scratch_types=[
        [pltpu.VMEM((WIN,), jnp.int32)] * NBUF,
        [pltpu.VMEM((WIN,), jnp.int32)] * NBUF,
        [pltpu.VMEM((WIN, D), jnp.float32)] * NBUF,
        pltpu.VMEM((TAIL,), jnp.int32),
        pltpu.VMEM((TAIL,), jnp.int32),
        pltpu.VMEM((TAIL, D), jnp.float32),
        pltpu.VMEM_SHARED((N, D), jnp.float32),
        [pltpu.SemaphoreType.DMA] * NBUF,
        [pltpu.SemaphoreType.DMA] * NBUF,
        [pltpu.SemaphoreType.DMA] * NBUF,
    ],
)
def _spmm_sc(src_hbm, dst_hbm, g_hbm, zrows_hbm, parts_out,
             src_v, dst_v, rows_v, src_t, dst_t, rows_t, acc, si, sg, ss):
    c = lax.axis_index("c")
    s = lax.axis_index("s")

    @pl.when(c == 0)
    def _():
        _chunked_row_copy(s, lambda r, n: pltpu.sync_copy(g_hbm.at[r], acc.at[r]))

    @pl.when(c != 0)
    def _():
        _chunked_row_copy(s, lambda r, n: pltpu.sync_copy(zrows_hbm.at[r], acc.at[r]))

    plsc.subcore_barrier()
    tile_base = (c * NS + s) * EDGES_PER_TILE

    def win_slice(w):
        return pl.ds(pl.multiple_of(tile_base + w * WIN, 8), WIN)

    def idx_start(w, b):
        pltpu.async_copy(src_hbm.at[win_slice(w)], src_v[b], si[b])
        pltpu.async_copy(dst_hbm.at[win_slice(w)], dst_v[b], si[b])

    def idx_wait(w, b):
        pltpu.make_async_copy(src_hbm.at[win_slice(w)], src_v[b], si[b]).wait()
        pltpu.make_async_copy(dst_hbm.at[win_slice(w)], dst_v[b], si[b]).wait()

    def gather_start(b):
        pltpu.async_copy(g_hbm.at[src_v[b]], rows_v[b], sg[b])

    def gather_wait(b):
        pltpu.make_async_copy(g_hbm.at[src_v[b]], rows_v[b], sg[b]).wait()

    def scatter_start(b):
        pltpu.async_copy(rows_v[b], acc.at[dst_v[b]], ss[b], add=True)

    def scatter_wait(b):
        pltpu.make_async_copy(rows_v[b], acc.at[dst_v[b]], ss[b]).wait()

    idx_start(0, 0)

    def body(i, carry):
        b = lax.rem(i, NBUF)
        bn = lax.rem(i + 1, NBUF)
        bp = lax.rem(i + NBUF - 1, NBUF)

        def at(bufsel, fn):
            for k in range(NBUF):
                pl.when(bufsel == k)(lambda kk=k: fn(kk))

        @pl.when(i >= NBUF - 1)
        def _():
            at(bn, scatter_wait)

        @pl.when(i <= NWIN - 2)
        def _():
            for k in range(NBUF):
                pl.when(bn == k)(lambda kk=k: idx_start(i + 1, kk))

        @pl.when(i >= 1)
        def _():
            at(bp, gather_wait)
            at(bp, scatter_start)

        for k in range(NBUF):
            pl.when(b == k)(lambda kk=k: idx_wait(i, kk))
        at(b, gather_start)
        return carry

    lax.fori_loop(0, NWIN, body, 0)

    lb = (NWIN - 1) % NBUF
    gather_wait(lb)
    scatter_start(lb)
    for j in range(max(NWIN - NBUF + 1, 0), NWIN):
        scatter_wait(j % NBUF)

    tsl = pl.ds(pl.multiple_of(tile_base + TAIL_OFF, 8), TAIL)
    pltpu.sync_copy(src_hbm.at[tsl], src_t)
    pltpu.sync_copy(dst_hbm.at[tsl], dst_t)
    pltpu.async_copy(g_hbm.at[src_t], rows_t, si[0]).wait()
    pltpu.sync_copy(rows_t, acc.at[dst_t], add=True)

    plsc.subcore_barrier()
    _chunked_row_copy(s, lambda r, n: pltpu.sync_copy(acc.at[r], parts_out.at[c, r]))


def _stage1_body(x_ref, w_ref, deg0_ref, deg1_ref, g_ref, dinv_ref):
    d = deg0_ref[...] + deg1_ref[...] + 1.0
    dinv = lax.rsqrt(d)
    h = jnp.dot(x_ref[...], w_ref[...], preferred_element_type=jnp.float32)
    g_ref[...] = h * dinv
    dinv_ref[...] = dinv


def _stage1(x, W1, deg0, deg1):
    return pl.pallas_call(
        _stage1_body,
        grid=(GRID,),
        in_specs=[
            pl.BlockSpec((BN, D), lambda i: (i, 0)),
            pl.BlockSpec((D, D), lambda i: (0, 0)),
            pl.BlockSpec((BN, 1), lambda i: (i, 0)),
            pl.BlockSpec((BN, 1), lambda i: (i, 0)),
        ],
        out_specs=[
            pl.BlockSpec((BN, D), lambda i: (i, 0)),
            pl.BlockSpec((BN, 1), lambda i: (i, 0)),
        ],
        out_shape=[
            jax.ShapeDtypeStruct((N, D), jnp.float32),
            jax.ShapeDtypeStruct((N, 1), jnp.float32),
        ],
    )(x, W1, deg0, deg1)


def _stage2_body(parts_ref, dinv_ref, w_ref, b_ref, g_ref):
    o = dinv_ref[...] * (parts_ref[0] + parts_ref[1]) + b_ref[...]
    h = jnp.dot(o, w_ref[...], preferred_element_type=jnp.float32)
    g_ref[...] = h * dinv_ref[...]


def _stage2(parts, dinv, W2, b1r):
    return pl.pallas_call(
        _stage2_body,
        grid=(GRID,),
        in_specs=[
            pl.BlockSpec((NC, BN, D), lambda i: (0, i, 0)),
            pl.BlockSpec((BN, 1), lambda i: (i, 0)),
            pl.BlockSpec((D, D), lambda i: (0, 0)),
            pl.BlockSpec((1, D), lambda i: (0, 0)),
        ],
        out_specs=pl.BlockSpec((BN, D), lambda i: (i, 0)),
        out_shape=jax.ShapeDtypeStruct((N, D), jnp.float32),
    )(parts, dinv, W2, b1r)


def _stage3_body(parts_ref, dinv_ref, b_ref, out_ref):
    o = dinv_ref[...] * (parts_ref[0] + parts_ref[1]) + b_ref[...]
    m = jnp.max(o, axis=1, keepdims=True)
    ex = jnp.exp(o - m)
    lse = jnp.log(jnp.sum(ex, axis=1, keepdims=True))
    out_ref[...] = o - m - lse


def _stage3(parts, dinv, b2r):
    return pl.pallas_call(
        _stage3_body,
        grid=(GRID,),
        in_specs=[
            pl.BlockSpec((NC, BN, D), lambda i: (0, i, 0)),
            pl.BlockSpec((BN, 1), lambda i: (i, 0)),
            pl.BlockSpec((1, D), lambda i: (0, 0)),
        ],
        out_specs=pl.BlockSpec((BN, D), lambda i: (i, 0)),
        out_shape=jax.ShapeDtypeStruct((N, D), jnp.float32),
    )(parts, dinv, b2r)


def kernel(x, edge_index, W1, b1, W2, b2, original_size):
    ones_win = jnp.ones((WIN,), jnp.float32)
    z1d = jnp.zeros((N,), jnp.float32)
    zrows = jnp.zeros((N, D), jnp.float32)
    src = edge_index[0]
    dst = edge_index[1]

    deg0, deg1 = _deg_sc(dst, ones_win, z1d)
    g1, dinv = _stage1(x, W1, jnp.reshape(deg0, (N, 1)), jnp.reshape(deg1, (N, 1)))
    parts1 = _spmm_sc(src, dst, g1, zrows)
    g2 = _stage2(parts1, dinv, W2, jnp.reshape(b1, (1, D)))
    parts2 = _spmm_sc(src, dst, g2, zrows)
    out = _stage3(parts2, dinv, jnp.reshape(b2, (1, D)))
    return out
